# Initial kernel scaffold; baseline (speedup 1.0000x reference)
import jax
import jax.numpy as jnp
from jax.experimental import pallas as pl


def kernel(chosen_types, song_tokens, scores):
    raise NotImplementedError("write your pallas kernel here")


# speedup vs baseline: 723.7177x; 723.7177x over previous
import jax
import jax.numpy as jnp
from jax.experimental import pallas as pl
from jax.experimental.pallas import tpu as pltpu

_L = 2047
_TOTAL = 1391
_INST_OFF = 1133
_INST_LEN = 129
_B = 4
_T = 256
_NT = 8
_LP = _NT * _T
_NEG = -1e9


def _mask_kernel(ev0_ref, id0_ref, ct_ref, ev_ref, id_ref, s_ref, o_ref,
                 carry_ref):
    b = pl.program_id(0)
    t = pl.program_id(1)

    @pl.when(t == 0)
    def _():
        lane0 = jax.lax.broadcasted_iota(jnp.int32, (1, _INST_LEN), 1)
        hit0 = (lane0 == id0_ref[b]).astype(jnp.float32)
        ev0 = (ev0_ref[b] == 1).astype(jnp.float32)
        carry_ref[...] = hit0 * ev0

    ct = ct_ref[0]
    ev = ev_ref[0]
    ids = id_ref[0]
    s = s_ref[0]

    lane = jax.lax.broadcasted_iota(jnp.int32, (_T, _INST_LEN), 1)
    hit = (lane == ids).astype(jnp.float32)
    e = (hit * (ev == 1).astype(jnp.float32)).astype(jnp.bfloat16)

    row = jax.lax.broadcasted_iota(jnp.int32, (_T, _T), 0)
    col = jax.lax.broadcasted_iota(jnp.int32, (_T, _T), 1)
    tri = (row >= col).astype(jnp.float32).astype(jnp.bfloat16)

    cum = jax.lax.dot_general(tri, e, (((1,), (0,)), ((), ())),
                              preferred_element_type=jnp.float32)
    tot = cum + carry_ref[...]
    carry_ref[...] = tot[_T - 1:_T, :]

    present = (tot > 0.0).astype(jnp.float32)
    is_default = ((ct == 0) | (ct == 2) | (ct == 7)).astype(jnp.float32)
    is_inst = (ct == 1).astype(jnp.float32)
    suppress = is_default + is_inst * present

    o_ref[0] = s
    inst = s[:, _INST_OFF:_INST_OFF + _INST_LEN]
    o_ref[0, :, _INST_OFF:_INST_OFF + _INST_LEN] = jnp.where(
        suppress > 0.0, jnp.full_like(inst, _NEG), inst)


def kernel(chosen_types, song_tokens, scores):
    ct = chosen_types.astype(jnp.int32)
    song = song_tokens.astype(jnp.int32)
    ev_full = (song[:, :, 0] == 1).astype(jnp.int32)
    ids_full = song[:, :, 6]

    zpad = jnp.zeros((_B, _LP - _L + 1), jnp.int32)
    evs = jnp.concatenate([ev_full[:, 1:], zpad], axis=1)
    ids_s = jnp.concatenate([ids_full[:, 1:], zpad], axis=1)
    ctp = jnp.concatenate([ct, zpad[:, :_LP - _L]], axis=1)
    ev0 = ev_full[:, 0]
    id0 = ids_full[:, 0]

    return pl.pallas_call(
        _mask_kernel,
        grid=(_B, _NT),
        in_specs=[
            pl.BlockSpec(memory_space=pltpu.SMEM),
            pl.BlockSpec(memory_space=pltpu.SMEM),
            pl.BlockSpec((1, _T, 1), lambda b, t: (b, t, 0)),
            pl.BlockSpec((1, _T, 1), lambda b, t: (b, t, 0)),
            pl.BlockSpec((1, _T, 1), lambda b, t: (b, t, 0)),
            pl.BlockSpec((1, _T, _TOTAL), lambda b, t: (b, t, 0)),
        ],
        out_specs=pl.BlockSpec((1, _T, _TOTAL), lambda b, t: (b, t, 0)),
        out_shape=jax.ShapeDtypeStruct((_B, _L, _TOTAL), jnp.float32),
        scratch_shapes=[pltpu.VMEM((1, _INST_LEN), jnp.float32)],
        compiler_params=pltpu.CompilerParams(
            dimension_semantics=("parallel", "arbitrary")),
    )(ev0, id0,
      ctp.reshape(_B, _LP, 1), evs.reshape(_B, _LP, 1),
      ids_s.reshape(_B, _LP, 1), scores)
